# baseline (speedup 1.0000x reference)
import dataclasses
import functools

import jax
import jax.numpy as jnp
from jax import lax
from jax.experimental import pallas as pl
from jax.experimental.pallas import tpu as pltpu
from jax.experimental.pallas import tpu_sc as plsc

B = 8192
F = 128
L = 16
GROUPS = F // L
CW = 9 * F
VW = 3 * F
CB = 8
EPS2 = 1e-24


def _rsqrt(s):
    i = lax.bitcast_convert_type(s, jnp.int32)
    i = jnp.int32(0x5F3759DF) - lax.shift_right_logical(i, 1)
    y = lax.bitcast_convert_type(i, jnp.float32)
    sh = 0.5 * s
    for _ in range(3):
        y = y * (1.5 - sh * y * y)
    return y


def _nvl_block(co_v, cg_v, va_v, out_v):
    lane9 = lax.iota(jnp.int32, L) * 9
    lane3 = lax.iota(jnp.int32, L) * 3

    @pl.loop(0, CB)
    def _row(b):
        cb = b * CW + lane9
        vb = b * VW + lane3

        @pl.loop(0, GROUPS)
        def _group(g):
            base9 = cb + g * (9 * L)
            base3 = vb + g * (3 * L)

            def ld(ref, base, off):
                return plsc.load_gather(ref, [base + off])

            ox0, oy0, oz0 = ld(co_v, base9, 0), ld(co_v, base9, 1), ld(co_v, base9, 2)
            ox1, oy1, oz1 = ld(co_v, base9, 3), ld(co_v, base9, 4), ld(co_v, base9, 5)
            ox2, oy2, oz2 = ld(co_v, base9, 6), ld(co_v, base9, 7), ld(co_v, base9, 8)
            a1x, a1y, a1z = ox1 - ox0, oy1 - oy0, oz1 - oz0
            a2x, a2y, a2z = ox2 - ox0, oy2 - oy0, oz2 - oz0
            a3x, a3y, a3z = a2x - a1x, a2y - a1y, a2z - a1z
            r1 = _rsqrt(jnp.maximum(a1x * a1x + a1y * a1y + a1z * a1z, EPS2))
            r2 = _rsqrt(jnp.maximum(a2x * a2x + a2y * a2y + a2z * a2z, EPS2))
            r3 = _rsqrt(jnp.maximum(a3x * a3x + a3y * a3y + a3z * a3z, EPS2))

            gx0, gy0, gz0 = ld(cg_v, base9, 0), ld(cg_v, base9, 1), ld(cg_v, base9, 2)
            gx1, gy1, gz1 = ld(cg_v, base9, 3), ld(cg_v, base9, 4), ld(cg_v, base9, 5)
            gx2, gy2, gz2 = ld(cg_v, base9, 6), ld(cg_v, base9, 7), ld(cg_v, base9, 8)
            e1x, e1y, e1z = gx1 - gx0, gy1 - gy0, gz1 - gz0
            e2x, e2y, e2z = gx2 - gx0, gy2 - gy0, gz2 - gz0
            re1 = _rsqrt(jnp.maximum(e1x * e1x + e1y * e1y + e1z * e1z, EPS2))
            re2 = _rsqrt(jnp.maximum(e2x * e2x + e2y * e2y + e2z * e2z, EPS2))
            q = re1 * re2
            cx = e1y * e2z - e1z * e2y
            cy = e1z * e2x - e1x * e2z
            cz = e1x * e2y - e1y * e2x
            sc = (cx * cx + cy * cy + cz * cz) * q * q
            t = q * _rsqrt(jnp.maximum(sc, EPS2))
            nx, ny, nz = cx * t, cy * t, cz * t

            m = ld(va_v, base3, 0) * ld(va_v, base3, 1) * ld(va_v, base3, 2)
            m1, m2, m3 = m * r1, m * r2, m * r3
            cos1 = jnp.abs(a1x * nx + a1y * ny + a1z * nz) * m1
            cos2 = jnp.abs(a2x * nx + a2y * ny + a2z * nz) * m2
            cos3 = jnp.abs(a3x * nx + a3y * ny + a3z * nz) * m3

            obase = b * VW + g * L
            out_v[pl.ds(obase, L)] = cos1
            out_v[pl.ds(obase + F, L)] = cos2
            out_v[pl.ds(obase + 2 * F, L)] = cos3


@jax.jit
def _nvl(co, cg, va):
    mesh = plsc.VectorSubcoreMesh(core_axis_name="core",
                                  subcore_axis_name="subcore")
    cp = pltpu.CompilerParams()
    if "needs_layout_passes" in pltpu.CompilerParams.__dataclass_fields__:
        cp = dataclasses.replace(cp, needs_layout_passes=False)

    @functools.partial(
        pl.kernel,
        out_type=jax.ShapeDtypeStruct((B * VW,), jnp.float32),
        mesh=mesh,
        compiler_params=cp,
    )
    def knl(co_hbm, cg_hbm, va_hbm, out_hbm):
        pltpu.emit_pipeline(
            _nvl_block,
            grid=(B // CB,),
            in_specs=[
                pl.BlockSpec((CB * CW,), lambda i: (i,)),
                pl.BlockSpec((CB * CW,), lambda i: (i,)),
                pl.BlockSpec((CB * VW,), lambda i: (i,)),
            ],
            out_specs=[pl.BlockSpec((CB * VW,), lambda i: (i,))],
            core_axis_name=("core", "subcore"),
            dimension_semantics=(pltpu.PARALLEL,),
        )(co_hbm, cg_hbm, va_hbm, out_hbm)

    return knl(co, cg, va)


def kernel(coord_out, coord_gt, valid):
    co = coord_out.reshape(B * CW)
    cg = coord_gt.reshape(B * CW)
    va = valid.reshape(B * VW)
    return _nvl(co, cg, va).reshape(B, VW, 1)


# speedup vs baseline: 133.5330x; 133.5330x over previous
import dataclasses
import functools

import jax
import jax.numpy as jnp
from jax import lax
from jax.experimental import pallas as pl
from jax.experimental.pallas import tpu as pltpu
from jax.experimental.pallas import tpu_sc as plsc

B = 8192
F = 128
L = 16
GROUPS = F // L
VW = 3 * F
CB = 8
EPS2 = 1e-24


def _rsqrt(s):
    i = lax.bitcast_convert_type(s, jnp.int32)
    i = jnp.int32(0x5F3759DF) - lax.shift_right_logical(i, 1)
    y = lax.bitcast_convert_type(i, jnp.float32)
    sh = 0.5 * s
    for _ in range(3):
        y = y * (1.5 - sh * y * y)
    return y


def _nvl_block(co_v, cg_v, va_v, out_v):
    lane = lax.iota(jnp.int32, L)
    lane3 = lane * 3
    zero16 = jnp.zeros((L,), jnp.int32)
    comp = [zero16, zero16 + 1, zero16 + 2]

    @pl.loop(0, CB)
    def _row(b):
        row = zero16 + b
        vflat = b * VW + lane3

        @pl.loop(0, GROUPS)
        def _group(g):
            vbase = lane3 + g * (3 * L)
            fbase3 = vflat + g * (3 * L)

            def ld(ref, dv, c):
                return plsc.load_gather(ref, [comp[c], row, vbase + dv])

            ox0, oy0, oz0 = ld(co_v, 0, 0), ld(co_v, 0, 1), ld(co_v, 0, 2)
            ox1, oy1, oz1 = ld(co_v, 1, 0), ld(co_v, 1, 1), ld(co_v, 1, 2)
            ox2, oy2, oz2 = ld(co_v, 2, 0), ld(co_v, 2, 1), ld(co_v, 2, 2)
            a1x, a1y, a1z = ox1 - ox0, oy1 - oy0, oz1 - oz0
            a2x, a2y, a2z = ox2 - ox0, oy2 - oy0, oz2 - oz0
            a3x, a3y, a3z = a2x - a1x, a2y - a1y, a2z - a1z
            r1 = _rsqrt(jnp.maximum(a1x * a1x + a1y * a1y + a1z * a1z, EPS2))
            r2 = _rsqrt(jnp.maximum(a2x * a2x + a2y * a2y + a2z * a2z, EPS2))
            r3 = _rsqrt(jnp.maximum(a3x * a3x + a3y * a3y + a3z * a3z, EPS2))

            gx0, gy0, gz0 = ld(cg_v, 0, 0), ld(cg_v, 0, 1), ld(cg_v, 0, 2)
            gx1, gy1, gz1 = ld(cg_v, 1, 0), ld(cg_v, 1, 1), ld(cg_v, 1, 2)
            gx2, gy2, gz2 = ld(cg_v, 2, 0), ld(cg_v, 2, 1), ld(cg_v, 2, 2)
            e1x, e1y, e1z = gx1 - gx0, gy1 - gy0, gz1 - gz0
            e2x, e2y, e2z = gx2 - gx0, gy2 - gy0, gz2 - gz0
            re1 = _rsqrt(jnp.maximum(e1x * e1x + e1y * e1y + e1z * e1z, EPS2))
            re2 = _rsqrt(jnp.maximum(e2x * e2x + e2y * e2y + e2z * e2z, EPS2))
            q = re1 * re2
            cx = e1y * e2z - e1z * e2y
            cy = e1z * e2x - e1x * e2z
            cz = e1x * e2y - e1y * e2x
            sc = (cx * cx + cy * cy + cz * cz) * q * q
            t = q * _rsqrt(jnp.maximum(sc, EPS2))
            nx, ny, nz = cx * t, cy * t, cz * t

            def lv(dv):
                return plsc.load_gather(va_v, [fbase3 + dv])

            m = lv(0) * lv(1) * lv(2)
            m1, m2, m3 = m * r1, m * r2, m * r3
            cos1 = jnp.abs(a1x * nx + a1y * ny + a1z * nz) * m1
            cos2 = jnp.abs(a2x * nx + a2y * ny + a2z * nz) * m2
            cos3 = jnp.abs(a3x * nx + a3y * ny + a3z * nz) * m3

            obase = b * VW + g * L
            out_v[pl.ds(obase, L)] = cos1
            out_v[pl.ds(obase + F, L)] = cos2
            out_v[pl.ds(obase + 2 * F, L)] = cos3


@jax.jit
def _nvl(co, cg, va):
    mesh = plsc.VectorSubcoreMesh(core_axis_name="core",
                                  subcore_axis_name="subcore")
    cp = pltpu.CompilerParams()
    if "needs_layout_passes" in pltpu.CompilerParams.__dataclass_fields__:
        cp = dataclasses.replace(cp, needs_layout_passes=False)
    cp = dataclasses.replace(cp, use_tc_tiling_on_sc=True)

    @functools.partial(
        pl.kernel,
        out_type=jax.ShapeDtypeStruct((B * VW,), jnp.float32),
        mesh=mesh,
        compiler_params=cp,
    )
    def knl(co_hbm, cg_hbm, va_hbm, out_hbm):
        pltpu.emit_pipeline(
            _nvl_block,
            grid=(B // CB,),
            in_specs=[
                pl.BlockSpec((3, CB, VW), lambda i: (0, i, 0)),
                pl.BlockSpec((3, CB, VW), lambda i: (0, i, 0)),
                pl.BlockSpec((CB * VW,), lambda i: (i,)),
            ],
            out_specs=[pl.BlockSpec((CB * VW,), lambda i: (i,))],
            core_axis_name=("core", "subcore"),
            dimension_semantics=(pltpu.PARALLEL,),
        )(co_hbm, cg_hbm, va_hbm, out_hbm)

    return knl(co, cg, va)


def kernel(coord_out, coord_gt, valid):
    co = jnp.transpose(coord_out, (2, 0, 1))
    cg = jnp.transpose(coord_gt, (2, 0, 1))
    va = valid.reshape(B * VW)
    return _nvl(co, cg, va).reshape(B, VW, 1)


# speedup vs baseline: 155.3778x; 1.1636x over previous
import dataclasses
import functools

import jax
import jax.numpy as jnp
from jax import lax
from jax.experimental import pallas as pl
from jax.experimental.pallas import tpu as pltpu
from jax.experimental.pallas import tpu_sc as plsc

B = 8192
F = 128
L = 16
GROUPS = F // L
VW = 3 * F
CB = 8
EPS2 = 1e-24


def _rsqrt(s):
    i = lax.bitcast_convert_type(s, jnp.int32)
    i = jnp.int32(0x5F3759DF) - lax.shift_right_logical(i, 1)
    y = lax.bitcast_convert_type(i, jnp.float32)
    sh = 0.5 * s
    for _ in range(2):
        y = y * (1.5 - sh * y * y)
    return y


def _nvl_block(co_v, cg_v, va_v, out_v):
    lane = lax.iota(jnp.int32, L)
    lane3 = lane * 3
    zero16 = jnp.zeros((L,), jnp.int32)
    comp = [zero16, zero16 + 1, zero16 + 2]

    @pl.loop(0, CB)
    def _row(b):
        row = zero16 + b
        vflat = b * VW + lane3

        @pl.loop(0, GROUPS, step=2)
        def _gpair(g0):
            for g in (g0, g0 + 1):
                _face_group(co_v, cg_v, va_v, out_v, comp, row, lane3,
                            vflat, b, g)


def _face_group(co_v, cg_v, va_v, out_v, comp, row, lane3, vflat, b, g):
            vbase = lane3 + g * (3 * L)
            fbase3 = vflat + g * (3 * L)

            def ld(ref, dv, c):
                return plsc.load_gather(ref, [comp[c], row, vbase + dv])

            ox0, oy0, oz0 = ld(co_v, 0, 0), ld(co_v, 0, 1), ld(co_v, 0, 2)
            ox1, oy1, oz1 = ld(co_v, 1, 0), ld(co_v, 1, 1), ld(co_v, 1, 2)
            ox2, oy2, oz2 = ld(co_v, 2, 0), ld(co_v, 2, 1), ld(co_v, 2, 2)
            a1x, a1y, a1z = ox1 - ox0, oy1 - oy0, oz1 - oz0
            a2x, a2y, a2z = ox2 - ox0, oy2 - oy0, oz2 - oz0
            a3x, a3y, a3z = a2x - a1x, a2y - a1y, a2z - a1z
            r1 = _rsqrt(jnp.maximum(a1x * a1x + a1y * a1y + a1z * a1z, EPS2))
            r2 = _rsqrt(jnp.maximum(a2x * a2x + a2y * a2y + a2z * a2z, EPS2))
            r3 = _rsqrt(jnp.maximum(a3x * a3x + a3y * a3y + a3z * a3z, EPS2))

            gx0, gy0, gz0 = ld(cg_v, 0, 0), ld(cg_v, 0, 1), ld(cg_v, 0, 2)
            gx1, gy1, gz1 = ld(cg_v, 1, 0), ld(cg_v, 1, 1), ld(cg_v, 1, 2)
            gx2, gy2, gz2 = ld(cg_v, 2, 0), ld(cg_v, 2, 1), ld(cg_v, 2, 2)
            e1x, e1y, e1z = gx1 - gx0, gy1 - gy0, gz1 - gz0
            e2x, e2y, e2z = gx2 - gx0, gy2 - gy0, gz2 - gz0
            re1 = _rsqrt(jnp.maximum(e1x * e1x + e1y * e1y + e1z * e1z, EPS2))
            re2 = _rsqrt(jnp.maximum(e2x * e2x + e2y * e2y + e2z * e2z, EPS2))
            q = re1 * re2
            cx = e1y * e2z - e1z * e2y
            cy = e1z * e2x - e1x * e2z
            cz = e1x * e2y - e1y * e2x
            sc = (cx * cx + cy * cy + cz * cz) * q * q
            t = q * _rsqrt(jnp.maximum(sc, EPS2))
            nx, ny, nz = cx * t, cy * t, cz * t

            def lv(dv):
                return plsc.load_gather(va_v, [fbase3 + dv])

            m = lv(0) * lv(1) * lv(2)
            m1, m2, m3 = m * r1, m * r2, m * r3
            cos1 = jnp.abs(a1x * nx + a1y * ny + a1z * nz) * m1
            cos2 = jnp.abs(a2x * nx + a2y * ny + a2z * nz) * m2
            cos3 = jnp.abs(a3x * nx + a3y * ny + a3z * nz) * m3

            obase = b * VW + g * L
            out_v[pl.ds(obase, L)] = cos1
            out_v[pl.ds(obase + F, L)] = cos2
            out_v[pl.ds(obase + 2 * F, L)] = cos3


@jax.jit
def _nvl(co, cg, va):
    mesh = plsc.VectorSubcoreMesh(core_axis_name="core",
                                  subcore_axis_name="subcore")
    cp = pltpu.CompilerParams()
    if "needs_layout_passes" in pltpu.CompilerParams.__dataclass_fields__:
        cp = dataclasses.replace(cp, needs_layout_passes=False)
    cp = dataclasses.replace(cp, use_tc_tiling_on_sc=True)

    @functools.partial(
        pl.kernel,
        out_type=jax.ShapeDtypeStruct((B * VW,), jnp.float32),
        mesh=mesh,
        compiler_params=cp,
    )
    def knl(co_hbm, cg_hbm, va_hbm, out_hbm):
        pltpu.emit_pipeline(
            _nvl_block,
            grid=(B // CB,),
            in_specs=[
                pl.BlockSpec((3, CB, VW), lambda i: (0, i, 0)),
                pl.BlockSpec((3, CB, VW), lambda i: (0, i, 0)),
                pl.BlockSpec((CB * VW,), lambda i: (i,)),
            ],
            out_specs=[pl.BlockSpec((CB * VW,), lambda i: (i,))],
            core_axis_name=("core", "subcore"),
            dimension_semantics=(pltpu.PARALLEL,),
        )(co_hbm, cg_hbm, va_hbm, out_hbm)

    return knl(co, cg, va)


def kernel(coord_out, coord_gt, valid):
    co = jnp.transpose(coord_out, (2, 0, 1))
    cg = jnp.transpose(coord_gt, (2, 0, 1))
    va = valid.reshape(B * VW)
    return _nvl(co, cg, va).reshape(B, VW, 1)


# speedup vs baseline: 157.0672x; 1.0109x over previous
import dataclasses
import functools

import jax
import jax.numpy as jnp
from jax import lax
from jax.experimental import pallas as pl
from jax.experimental.pallas import tpu as pltpu
from jax.experimental.pallas import tpu_sc as plsc

B = 8192
F = 128
L = 16
GROUPS = F // L
VW = 3 * F
CB = 16
EPS2 = 1e-24


def _rsqrt(s):
    i = lax.bitcast_convert_type(s, jnp.int32)
    i = jnp.int32(0x5F3759DF) - lax.shift_right_logical(i, 1)
    y = lax.bitcast_convert_type(i, jnp.float32)
    sh = 0.5 * s
    for _ in range(2):
        y = y * (1.5 - sh * y * y)
    return y


def _nvl_block(co_v, cg_v, va_v, out_v):
    lane = lax.iota(jnp.int32, L)
    lane3 = lane * 3
    zero16 = jnp.zeros((L,), jnp.int32)
    comp = [zero16, zero16 + 1, zero16 + 2]

    @pl.loop(0, CB)
    def _row(b):
        row = zero16 + b
        vflat = b * VW + lane3

        @pl.loop(0, GROUPS, step=4)
        def _gpair(g0):
            for g in (g0, g0 + 1, g0 + 2, g0 + 3):
                _face_group(co_v, cg_v, va_v, out_v, comp, row, lane3,
                            vflat, b, g)


def _face_group(co_v, cg_v, va_v, out_v, comp, row, lane3, vflat, b, g):
            vbase = lane3 + g * (3 * L)
            fbase3 = vflat + g * (3 * L)

            def ld(ref, dv, c):
                return plsc.load_gather(ref, [comp[c], row, vbase + dv])

            ox0, oy0, oz0 = ld(co_v, 0, 0), ld(co_v, 0, 1), ld(co_v, 0, 2)
            ox1, oy1, oz1 = ld(co_v, 1, 0), ld(co_v, 1, 1), ld(co_v, 1, 2)
            ox2, oy2, oz2 = ld(co_v, 2, 0), ld(co_v, 2, 1), ld(co_v, 2, 2)
            a1x, a1y, a1z = ox1 - ox0, oy1 - oy0, oz1 - oz0
            a2x, a2y, a2z = ox2 - ox0, oy2 - oy0, oz2 - oz0
            a3x, a3y, a3z = a2x - a1x, a2y - a1y, a2z - a1z
            r1 = _rsqrt(jnp.maximum(a1x * a1x + a1y * a1y + a1z * a1z, EPS2))
            r2 = _rsqrt(jnp.maximum(a2x * a2x + a2y * a2y + a2z * a2z, EPS2))
            r3 = _rsqrt(jnp.maximum(a3x * a3x + a3y * a3y + a3z * a3z, EPS2))

            gx0, gy0, gz0 = ld(cg_v, 0, 0), ld(cg_v, 0, 1), ld(cg_v, 0, 2)
            gx1, gy1, gz1 = ld(cg_v, 1, 0), ld(cg_v, 1, 1), ld(cg_v, 1, 2)
            gx2, gy2, gz2 = ld(cg_v, 2, 0), ld(cg_v, 2, 1), ld(cg_v, 2, 2)
            e1x, e1y, e1z = gx1 - gx0, gy1 - gy0, gz1 - gz0
            e2x, e2y, e2z = gx2 - gx0, gy2 - gy0, gz2 - gz0
            re1 = _rsqrt(jnp.maximum(e1x * e1x + e1y * e1y + e1z * e1z, EPS2))
            re2 = _rsqrt(jnp.maximum(e2x * e2x + e2y * e2y + e2z * e2z, EPS2))
            q = re1 * re2
            cx = e1y * e2z - e1z * e2y
            cy = e1z * e2x - e1x * e2z
            cz = e1x * e2y - e1y * e2x
            sc = (cx * cx + cy * cy + cz * cz) * q * q
            t = q * _rsqrt(jnp.maximum(sc, EPS2))
            nx, ny, nz = cx * t, cy * t, cz * t

            def lv(dv):
                return plsc.load_gather(va_v, [fbase3 + dv])

            m = lv(0) * lv(1) * lv(2)
            m1, m2, m3 = m * r1, m * r2, m * r3
            cos1 = jnp.abs(a1x * nx + a1y * ny + a1z * nz) * m1
            cos2 = jnp.abs(a2x * nx + a2y * ny + a2z * nz) * m2
            cos3 = jnp.abs(a3x * nx + a3y * ny + a3z * nz) * m3

            obase = b * VW + g * L
            out_v[pl.ds(obase, L)] = cos1
            out_v[pl.ds(obase + F, L)] = cos2
            out_v[pl.ds(obase + 2 * F, L)] = cos3


@jax.jit
def _nvl(co, cg, va):
    mesh = plsc.VectorSubcoreMesh(core_axis_name="core",
                                  subcore_axis_name="subcore")
    cp = pltpu.CompilerParams()
    if "needs_layout_passes" in pltpu.CompilerParams.__dataclass_fields__:
        cp = dataclasses.replace(cp, needs_layout_passes=False)
    cp = dataclasses.replace(cp, use_tc_tiling_on_sc=True)

    @functools.partial(
        pl.kernel,
        out_type=jax.ShapeDtypeStruct((B * VW,), jnp.float32),
        mesh=mesh,
        compiler_params=cp,
    )
    def knl(co_hbm, cg_hbm, va_hbm, out_hbm):
        pltpu.emit_pipeline(
            _nvl_block,
            grid=(B // CB,),
            in_specs=[
                pl.BlockSpec((3, CB, VW), lambda i: (0, i, 0)),
                pl.BlockSpec((3, CB, VW), lambda i: (0, i, 0)),
                pl.BlockSpec((CB * VW,), lambda i: (i,)),
            ],
            out_specs=[pl.BlockSpec((CB * VW,), lambda i: (i,))],
            core_axis_name=("core", "subcore"),
            dimension_semantics=(pltpu.PARALLEL,),
        )(co_hbm, cg_hbm, va_hbm, out_hbm)

    return knl(co, cg, va)


def kernel(coord_out, coord_gt, valid):
    co = jnp.transpose(coord_out, (2, 0, 1))
    cg = jnp.transpose(coord_gt, (2, 0, 1))
    va = valid.reshape(B * VW)
    return _nvl(co, cg, va).reshape(B, VW, 1)


# speedup vs baseline: 180.1693x; 1.1471x over previous
import dataclasses
import functools

import jax
import jax.numpy as jnp
from jax import lax
from jax.experimental import pallas as pl
from jax.experimental.pallas import tpu as pltpu
from jax.experimental.pallas import tpu_sc as plsc

B = 8192
F = 128
L = 16
GROUPS = F // L
VW = 3 * F
CB = 16
EPS2 = 1e-24


def _rsqrt(s):
    i = lax.bitcast_convert_type(s, jnp.int32)
    i = jnp.int32(0x5F3759DF) - lax.shift_right_logical(i, 1)
    y = lax.bitcast_convert_type(i, jnp.float32)
    sh = 0.5 * s
    for _ in range(2):
        y = y * (1.5 - sh * y * y)
    return y


def _nvl_block(co_v, cg_v, va_v, out_v):
    lane = lax.iota(jnp.int32, L)
    lane3 = lane * 3
    zero16 = jnp.zeros((L,), jnp.int32)
    comp = [zero16, zero16 + 1, zero16 + 2]

    @pl.loop(0, GROUPS, step=2)
    def _group(g0):
        @pl.loop(0, CB, step=2)
        def _rows(b0):
            for g in (g0, g0 + 1):
                for b in (b0, b0 + 1):
                    _face_group(co_v, cg_v, va_v, out_v, comp,
                                zero16 + b, lane3, b * VW + lane3, b, g)


def _face_group(co_v, cg_v, va_v, out_v, comp, row, lane3, vflat, b, g):
            vbase = lane3 + g * (3 * L)
            fbase3 = vflat + g * (3 * L)

            def ld(ref, dv, c):
                return plsc.load_gather(ref, [comp[c], row, vbase + dv])

            ox0, oy0, oz0 = ld(co_v, 0, 0), ld(co_v, 0, 1), ld(co_v, 0, 2)
            ox1, oy1, oz1 = ld(co_v, 1, 0), ld(co_v, 1, 1), ld(co_v, 1, 2)
            ox2, oy2, oz2 = ld(co_v, 2, 0), ld(co_v, 2, 1), ld(co_v, 2, 2)
            a1x, a1y, a1z = ox1 - ox0, oy1 - oy0, oz1 - oz0
            a2x, a2y, a2z = ox2 - ox0, oy2 - oy0, oz2 - oz0
            a3x, a3y, a3z = a2x - a1x, a2y - a1y, a2z - a1z
            r1 = _rsqrt(jnp.maximum(a1x * a1x + a1y * a1y + a1z * a1z, EPS2))
            r2 = _rsqrt(jnp.maximum(a2x * a2x + a2y * a2y + a2z * a2z, EPS2))
            r3 = _rsqrt(jnp.maximum(a3x * a3x + a3y * a3y + a3z * a3z, EPS2))

            gx0, gy0, gz0 = ld(cg_v, 0, 0), ld(cg_v, 0, 1), ld(cg_v, 0, 2)
            gx1, gy1, gz1 = ld(cg_v, 1, 0), ld(cg_v, 1, 1), ld(cg_v, 1, 2)
            gx2, gy2, gz2 = ld(cg_v, 2, 0), ld(cg_v, 2, 1), ld(cg_v, 2, 2)
            e1x, e1y, e1z = gx1 - gx0, gy1 - gy0, gz1 - gz0
            e2x, e2y, e2z = gx2 - gx0, gy2 - gy0, gz2 - gz0
            cx = e1y * e2z - e1z * e2y
            cy = e1z * e2x - e1x * e2z
            cz = e1x * e2y - e1y * e2x
            sc = cx * cx + cy * cy + cz * cz
            t = _rsqrt(jnp.maximum(sc, EPS2))
            nx, ny, nz = cx * t, cy * t, cz * t

            def lv(dv):
                return plsc.load_gather(va_v, [fbase3 + dv])

            m = lv(0) * lv(1) * lv(2)
            m1, m2, m3 = m * r1, m * r2, m * r3
            cos1 = jnp.abs(a1x * nx + a1y * ny + a1z * nz) * m1
            cos2 = jnp.abs(a2x * nx + a2y * ny + a2z * nz) * m2
            cos3 = jnp.abs(a3x * nx + a3y * ny + a3z * nz) * m3

            obase = b * VW + g * L
            out_v[pl.ds(obase, L)] = cos1
            out_v[pl.ds(obase + F, L)] = cos2
            out_v[pl.ds(obase + 2 * F, L)] = cos3


@jax.jit
def _nvl(co, cg, va):
    mesh = plsc.VectorSubcoreMesh(core_axis_name="core",
                                  subcore_axis_name="subcore")
    cp = pltpu.CompilerParams()
    if "needs_layout_passes" in pltpu.CompilerParams.__dataclass_fields__:
        cp = dataclasses.replace(cp, needs_layout_passes=False)
    cp = dataclasses.replace(cp, use_tc_tiling_on_sc=True)

    @functools.partial(
        pl.kernel,
        out_type=jax.ShapeDtypeStruct((B * VW,), jnp.float32),
        mesh=mesh,
        compiler_params=cp,
    )
    def knl(co_hbm, cg_hbm, va_hbm, out_hbm):
        pltpu.emit_pipeline(
            _nvl_block,
            grid=(B // CB,),
            in_specs=[
                pl.BlockSpec((3, CB, VW), lambda i: (0, i, 0)),
                pl.BlockSpec((3, CB, VW), lambda i: (0, i, 0)),
                pl.BlockSpec((CB * VW,), lambda i: (i,)),
            ],
            out_specs=[pl.BlockSpec((CB * VW,), lambda i: (i,))],
            core_axis_name=("core", "subcore"),
            dimension_semantics=(pltpu.PARALLEL,),
        )(co_hbm, cg_hbm, va_hbm, out_hbm)

    return knl(co, cg, va)


def kernel(coord_out, coord_gt, valid):
    co = jnp.transpose(coord_out, (2, 0, 1))
    cg = jnp.transpose(coord_gt, (2, 0, 1))
    va = valid.reshape(B * VW)
    return _nvl(co, cg, va).reshape(B, VW, 1)
